# baseline (speedup 1.0000x reference)
import functools
import math

import jax
import jax.numpy as jnp
from jax import lax
from jax.experimental import pallas as pl
from jax.experimental.pallas import tpu as pltpu
from jax.experimental.pallas import tpu_sc as plsc

MUL = 128
HID = 64
NATTR = 10
NPASS = 5
GW = 2 * HID
AVG_NUM_NEIGHBORS = 32.0
_L_OF_D = (0, 1, 1, 1, 2, 2, 2, 2, 2)


def _mlp_body(ed_ref, w0_ref, w1_ref, w2_ref, h_ref):
    h = jax.nn.silu(jnp.dot(ed_ref[...], w0_ref[...],
                            preferred_element_type=jnp.float32) * (1.0 / math.sqrt(8.0)))
    h = jax.nn.silu(jnp.dot(h, w1_ref[...],
                            preferred_element_type=jnp.float32) * (1.0 / math.sqrt(HID)))
    h_ref[...] = jax.nn.silu(jnp.dot(h, w2_ref[...],
                                     preferred_element_type=jnp.float32) * (1.0 / math.sqrt(HID)))


def _build_sc_scatter(EP, N):
    n_tiles = 32
    ept = EP // n_tiles
    C = 64
    n_chunks = ept // C
    NBUF = 4
    n_outer = n_chunks // NBUF
    rows = (N + 127) // 128 * 128 + 128
    rows_per_tile = rows // 16
    mesh = plsc.VectorSubcoreMesh(core_axis_name="c", subcore_axis_name="s")
    out_t = jax.ShapeDtypeStruct((NPASS, 2, rows, GW), jnp.float32)

    @functools.partial(
        pl.kernel, out_type=out_t, mesh=mesh,
        compiler_params=pltpu.CompilerParams(needs_layout_passes=False),
        scratch_types=[pltpu.VMEM((C, GW), jnp.float32)] * 2
                      + [pltpu.VMEM((C,), jnp.int32)] * 2
                      + [pltpu.VMEM((C // 2, 128), jnp.float32)] * NBUF
                      + [pltpu.VMEM((C,), jnp.int32)] * NBUF
                      + [pltpu.VMEM((C,), jnp.float32)] * (2 * NBUF)
                      + [pltpu.VMEM_SHARED((rows, GW), jnp.float32)]
                      + [pltpu.SemaphoreType.DMA] * (NBUF + 2))
    def sc_scatter(src_hbm, h_hbm, sh_hbm, zero_hbm, o_hbm, *rest):
        pays = rest[:2]
        sidx = rest[2:4]
        hbufs = rest[4:4 + NBUF]
        idxs = rest[4 + NBUF:4 + 2 * NBUF]
        sh0s = rest[4 + 2 * NBUF:4 + 3 * NBUF]
        sh1s = rest[4 + 3 * NBUF:4 + 4 * NBUF]
        accum = rest[4 + 4 * NBUF]
        gsems = rest[5 + 4 * NBUF:5 + 5 * NBUF]
        ssems = rest[5 + 5 * NBUF:]
        c = lax.axis_index("c")
        s = lax.axis_index("s")
        wid = c * 16 + s
        row0 = s * rows_per_tile

        def pass_body(p, pcarry):
            pltpu.sync_copy(zero_hbm.at[pl.ds(row0, rows_per_tile)],
                            accum.at[pl.ds(row0, rows_per_tile)])
            plsc.subcore_barrier()

            def fetch(i, b):
                pltpu.async_copy(src_hbm.at[wid, i], idxs[b], gsems[b])
                pltpu.async_copy(
                    h_hbm.at[pl.ds(wid * (ept // 2) + i * (C // 2), C // 2)],
                    hbufs[b], gsems[b])
                pltpu.async_copy(sh_hbm.at[2 * p, wid, i], sh0s[b], gsems[b])
                pltpu.async_copy(sh_hbm.at[2 * p + 1, wid, i], sh1s[b], gsems[b])

            for b in range(NBUF):
                fetch(b, b)

            def compute(b, pb):
                for v in range(C // 16):
                    sidx[pb][pl.ds(16 * v, 16)] = idxs[b][pl.ds(16 * v, 16)]

                def group16(eg, carry):
                    e0 = pl.multiple_of(16 * eg, 16)
                    r0 = pl.multiple_of(8 * eg, 8)
                    sv0 = sh0s[b][pl.ds(e0, 16)]
                    sv1 = sh1s[b][pl.ds(e0, 16)]
                    for k in range(16):
                        e = e0 + k
                        b0 = jnp.full((16,), sv0[k])
                        b1 = jnp.full((16,), sv1[k])
                        for j in range(HID // 16):
                            hv = hbufs[b][r0 + k // 2,
                                          pl.ds(HID * (k % 2) + 16 * j, 16)]
                            pays[pb][e, pl.ds(16 * j, 16)] = hv * b0
                            pays[pb][e, pl.ds(HID + 16 * j, 16)] = hv * b1
                    return carry
                lax.fori_loop(0, C // 16, group16, 0)

            def outer(j, carry):
                for b in range(NBUF):
                    i = j * NBUF + b
                    pb = b % 2
                    pltpu.make_async_copy(src_hbm.at[0, 0], idxs[b],
                                          gsems[b]).wait()
                    pltpu.make_async_copy(h_hbm.at[pl.ds(0, C // 2)],
                                          hbufs[b], gsems[b]).wait()
                    pltpu.make_async_copy(sh_hbm.at[0, 0, 0], sh0s[b],
                                          gsems[b]).wait()
                    pltpu.make_async_copy(sh_hbm.at[0, 0, 0], sh1s[b],
                                          gsems[b]).wait()
                    if b < 2:
                        @pl.when(j > 0)
                        def _():
                            pltpu.make_async_copy(pays[pb], accum.at[sidx[pb]],
                                                  ssems[pb]).wait()
                    else:
                        pltpu.make_async_copy(pays[pb], accum.at[sidx[pb]],
                                              ssems[pb]).wait()
                    compute(b, pb)
                    pltpu.async_copy(pays[pb], accum.at[sidx[pb]],
                                     ssems[pb], add=True)

                    @pl.when(j < n_outer - 1)
                    def _():
                        fetch(i + NBUF, b)
                return carry

            lax.fori_loop(0, n_outer, outer, 0)
            for pb in range(2):
                pltpu.make_async_copy(pays[pb], accum.at[sidx[pb]],
                                      ssems[pb]).wait()
            plsc.subcore_barrier()
            pltpu.sync_copy(accum.at[pl.ds(row0, rows_per_tile)],
                            o_hbm.at[p, c, pl.ds(row0, rows_per_tile)])
            return pcarry

        lax.fori_loop(0, NPASS, pass_body, 0)

    return sc_scatter


def _final_body(nf_ref, na_ref, *rest):
    t_refs = rest[:2 * NPASS]
    (wskip_ref, wlin_ref, wmlp3_ref, w20_ref, w21_ref, w22_ref,
     out9_ref, sc_ref) = rest[2 * NPASS:]
    nf = nf_ref[...]
    acc = jnp.zeros_like(nf)
    for v in range(NATTR):
        acc = acc + jnp.dot(nf, wskip_ref[:, v, :],
                            preferred_element_type=jnp.float32) * na_ref[:, v:v + 1]
    sc_ref[...] = acc * (1.0 / math.sqrt(MUL * NATTR))
    x = jnp.dot(nf, wlin_ref[...],
                preferred_element_type=jnp.float32) * (1.0 / math.sqrt(MUL))
    w2s = (w20_ref, w21_ref, w22_ref)
    scale = 1.0 / (math.sqrt(HID) * math.sqrt(MUL) * AVG_NUM_NEIGHBORS)
    for d in range(9):
        p, dl = divmod(d, 2)
        l = _L_OF_D[d]
        ta, tb = t_refs[2 * p], t_refs[2 * p + 1]
        td = (ta[0, 0, :, HID * dl:HID * (dl + 1)]
              + tb[0, 0, :, HID * dl:HID * (dl + 1)])
        m = jnp.dot(td, wmlp3_ref[:, l * MUL:(l + 1) * MUL],
                    preferred_element_type=jnp.float32)
        out9_ref[d] = jnp.dot(x * m, w2s[l][...],
                              preferred_element_type=jnp.float32) * scale


def kernel(node_feat, node_attr, edge_idx, edge_dist_embedding, edge_diff_embedding,
           W_skip, W_lin1, W_mlp0, W_mlp1, W_mlp2, W_mlp3, W2_0, W2_1, W2_2):
    N = node_feat.shape[0]
    E = edge_dist_embedding.shape[0]
    src = edge_idx[:, 0]
    rows = (N + 127) // 128 * 128 + 128

    EB = 8192
    EP = (E + EB - 1) // EB * EB
    ed_p = jnp.pad(edge_dist_embedding, ((0, EP - E), (0, 0)))
    h = pl.pallas_call(
        _mlp_body,
        grid=(EP // EB,),
        in_specs=[
            pl.BlockSpec((EB, 8), lambda i: (i, 0)),
            pl.BlockSpec((8, HID), lambda i: (0, 0)),
            pl.BlockSpec((HID, HID), lambda i: (0, 0)),
            pl.BlockSpec((HID, HID), lambda i: (0, 0)),
        ],
        out_specs=pl.BlockSpec((EB, HID), lambda i: (i, 0)),
        out_shape=jax.ShapeDtypeStruct((EP, HID), jnp.float32),
    )(ed_p, W_mlp0, W_mlp1, W_mlp2)

    zeros = jnp.zeros((rows, GW), jnp.float32)
    src_p = jnp.pad(src, (0, EP - E), constant_values=rows - 1)
    shp = jnp.pad(edge_diff_embedding, ((0, EP - E), (0, 1)))
    sh10 = shp.T.reshape(10, 32, -1, 64)
    t5 = _build_sc_scatter(EP, N)(src_p.reshape(32, -1, 64), h.reshape(-1, 128),
                                  sh10, zeros)

    NB = 400
    t_specs = []
    t_args = []
    for p in range(NPASS):
        for cc in range(2):
            t_specs.append(pl.BlockSpec(
                (1, 1, NB, GW), lambda i, p=p, cc=cc: (p, cc, i, 0)))
            t_args.append(t5)
    out9, sc = pl.pallas_call(
        _final_body,
        grid=(N // NB,),
        in_specs=[
            pl.BlockSpec((NB, MUL), lambda i: (i, 0)),
            pl.BlockSpec((NB, NATTR), lambda i: (i, 0)),
        ] + t_specs + [
            pl.BlockSpec((MUL, NATTR, MUL), lambda i: (0, 0, 0)),
            pl.BlockSpec((MUL, MUL), lambda i: (0, 0)),
            pl.BlockSpec((HID, 3 * MUL), lambda i: (0, 0)),
            pl.BlockSpec((MUL, MUL), lambda i: (0, 0)),
            pl.BlockSpec((MUL, MUL), lambda i: (0, 0)),
            pl.BlockSpec((MUL, MUL), lambda i: (0, 0)),
        ],
        out_specs=[
            pl.BlockSpec((9, NB, MUL), lambda i: (0, i, 0)),
            pl.BlockSpec((NB, MUL), lambda i: (i, 0)),
        ],
        out_shape=[
            jax.ShapeDtypeStruct((9, N, MUL), jnp.float32),
            jax.ShapeDtypeStruct((N, MUL), jnp.float32),
        ],
    )(node_feat, node_attr, *t_args,
      W_skip, W_lin1, W_mlp3, W2_0, W2_1, W2_2)

    return (jnp.transpose(out9, (1, 2, 0)), sc)


# speedup vs baseline: 1.1985x; 1.1985x over previous
import functools
import math

import jax
import jax.numpy as jnp
from jax import lax
from jax.experimental import pallas as pl
from jax.experimental.pallas import tpu as pltpu
from jax.experimental.pallas import tpu_sc as plsc

MUL = 128
HID = 64
NATTR = 10
NPASS = 5
GW = 2 * HID
AVG_NUM_NEIGHBORS = 32.0
_L_OF_D = (0, 1, 1, 1, 2, 2, 2, 2, 2)


def _mlp_g_body(ed_ref, sh_ref, w0_ref, w1_ref, w2_ref, *g_refs):
    h = jax.nn.silu(jnp.dot(ed_ref[...], w0_ref[...],
                            preferred_element_type=jnp.float32) * (1.0 / math.sqrt(8.0)))
    h = jax.nn.silu(jnp.dot(h, w1_ref[...],
                            preferred_element_type=jnp.float32) * (1.0 / math.sqrt(HID)))
    h = jax.nn.silu(jnp.dot(h, w2_ref[...],
                            preferred_element_type=jnp.float32) * (1.0 / math.sqrt(HID)))
    sh = sh_ref[...]
    for p, ref in enumerate(g_refs):
        cols = []
        for d in range(2):
            dc = 2 * p + d
            cols.append(sh[:, dc:dc + 1] * h if dc < 9 else jnp.zeros_like(h))
        ref[...] = jnp.concatenate(cols, axis=1)


def _build_sc_scatter(EP, N):
    n_tiles = 32
    ept = EP // n_tiles
    C = 64
    n_chunks = ept // C
    NBUF = 4
    n_outer = n_chunks // NBUF
    rows = (N + 127) // 128 * 128 + 128
    rows_per_tile = rows // 16
    mesh = plsc.VectorSubcoreMesh(core_axis_name="c", subcore_axis_name="s")
    out_t = tuple(jax.ShapeDtypeStruct((2, rows, GW), jnp.float32) for _ in range(NPASS))

    @functools.partial(
        pl.kernel, out_type=out_t, mesh=mesh,
        scratch_types=[pltpu.VMEM((C, GW), jnp.float32)] * NBUF
                      + [pltpu.VMEM((C,), jnp.int32)] * NBUF + [
            pltpu.VMEM_SHARED((rows, GW), jnp.float32),
        ] + [pltpu.SemaphoreType.DMA] * (2 * NBUF))
    def sc_scatter(src_hbm, g0_hbm, g1_hbm, g2_hbm, g3_hbm, g4_hbm, zero_hbm,
                   o0, o1, o2, o3, o4, *rest):
        bufs = rest[:NBUF]
        idxs = rest[NBUF:2 * NBUF]
        accum = rest[2 * NBUF]
        gsems = rest[2 * NBUF + 1:3 * NBUF + 1]
        ssems = rest[3 * NBUF + 1:]
        c = lax.axis_index("c")
        s = lax.axis_index("s")
        wid = c * 16 + s
        tile_base = wid * ept
        row0 = s * rows_per_tile

        for g_hbm, o_hbm in ((g0_hbm, o0), (g1_hbm, o1), (g2_hbm, o2),
                             (g3_hbm, o3), (g4_hbm, o4)):
            pltpu.sync_copy(zero_hbm.at[pl.ds(row0, rows_per_tile)],
                            accum.at[pl.ds(row0, rows_per_tile)])
            plsc.subcore_barrier()

            def fetch(i, b):
                pltpu.async_copy(src_hbm.at[wid, i], idxs[b], gsems[b])
                pltpu.async_copy(g_hbm.at[pl.ds(tile_base + i * C, C)],
                                 bufs[b], gsems[b])

            fetch(0, 0)
            fetch(1, 1)

            def outer(j, carry):
                for b in range(NBUF):
                    i = j * NBUF + b
                    b2 = (b + 2) % NBUF
                    if b < 2:
                        @pl.when(j > 0)
                        def _():
                            pltpu.make_async_copy(bufs[b2],
                                                  accum.at[idxs[b2]],
                                                  ssems[b2]).wait()
                        fetch(i + 2, b2)
                    else:
                        pltpu.make_async_copy(bufs[b2], accum.at[idxs[b2]],
                                              ssems[b2]).wait()

                        @pl.when(j < n_outer - 1)
                        def _():
                            fetch(i + 2, b2)
                    pltpu.make_async_copy(src_hbm.at[0, 0], idxs[b],
                                          gsems[b]).wait()
                    pltpu.make_async_copy(g_hbm.at[pl.ds(0, C)], bufs[b],
                                          gsems[b]).wait()
                    pltpu.async_copy(bufs[b], accum.at[idxs[b]],
                                     ssems[b], add=True)
                return carry

            lax.fori_loop(0, n_outer, outer, 0)
            for b in (2, 3):
                pltpu.make_async_copy(bufs[b], accum.at[idxs[b]],
                                      ssems[b]).wait()
            plsc.subcore_barrier()
            pltpu.sync_copy(accum.at[pl.ds(row0, rows_per_tile)],
                            o_hbm.at[c, pl.ds(row0, rows_per_tile)])

    return sc_scatter


def _final_body(nf_ref, na_ref, *rest):
    t_refs = rest[:2 * NPASS]
    (wskip_ref, wlin_ref, wmlp3_ref, w20_ref, w21_ref, w22_ref,
     out9_ref, sc_ref) = rest[2 * NPASS:]
    nf = nf_ref[...]
    acc = jnp.zeros_like(nf)
    for v in range(NATTR):
        acc = acc + jnp.dot(nf, wskip_ref[:, v, :],
                            preferred_element_type=jnp.float32) * na_ref[:, v:v + 1]
    sc_ref[...] = acc * (1.0 / math.sqrt(MUL * NATTR))
    x = jnp.dot(nf, wlin_ref[...],
                preferred_element_type=jnp.float32) * (1.0 / math.sqrt(MUL))
    w2s = (w20_ref, w21_ref, w22_ref)
    scale = 1.0 / (math.sqrt(HID) * math.sqrt(MUL) * AVG_NUM_NEIGHBORS)
    for d in range(9):
        p, dl = divmod(d, 2)
        l = _L_OF_D[d]
        ta, tb = t_refs[2 * p], t_refs[2 * p + 1]
        td = (ta[0, :, HID * dl:HID * (dl + 1)]
              + tb[0, :, HID * dl:HID * (dl + 1)])
        m = jnp.dot(td, wmlp3_ref[:, l * MUL:(l + 1) * MUL],
                    preferred_element_type=jnp.float32)
        out9_ref[d] = jnp.dot(x * m, w2s[l][...],
                              preferred_element_type=jnp.float32) * scale


def kernel(node_feat, node_attr, edge_idx, edge_dist_embedding, edge_diff_embedding,
           W_skip, W_lin1, W_mlp0, W_mlp1, W_mlp2, W_mlp3, W2_0, W2_1, W2_2):
    N = node_feat.shape[0]
    E = edge_dist_embedding.shape[0]
    src = edge_idx[:, 0]
    rows = (N + 127) // 128 * 128 + 128

    EP = (E + 8191) // 8192 * 8192
    EB = 4096
    ed_p = jnp.pad(edge_dist_embedding, ((0, EP - E), (0, 0)))
    sh_p = jnp.pad(edge_diff_embedding, ((0, EP - E), (0, 0)))
    gs = pl.pallas_call(
        _mlp_g_body,
        grid=(EP // EB,),
        in_specs=[
            pl.BlockSpec((EB, 8), lambda i: (i, 0)),
            pl.BlockSpec((EB, 9), lambda i: (i, 0)),
            pl.BlockSpec((8, HID), lambda i: (0, 0)),
            pl.BlockSpec((HID, HID), lambda i: (0, 0)),
            pl.BlockSpec((HID, HID), lambda i: (0, 0)),
        ],
        out_specs=[pl.BlockSpec((EB, GW), lambda i: (i, 0))] * NPASS,
        out_shape=[jax.ShapeDtypeStruct((EP, GW), jnp.float32)] * NPASS,
    )(ed_p, sh_p, W_mlp0, W_mlp1, W_mlp2)

    zeros = jnp.zeros((rows, GW), jnp.float32)
    src_p = jnp.pad(src, (0, EP - E), constant_values=rows - 1)
    ts = _build_sc_scatter(EP, N)(src_p.reshape(32, -1, 64), *gs, zeros)

    NB = 400
    t_specs = []
    t_args = []
    for t in ts:
        t_specs += [pl.BlockSpec((1, NB, GW), lambda i: (0, i, 0)),
                    pl.BlockSpec((1, NB, GW), lambda i: (1, i, 0))]
        t_args += [t, t]
    out9, sc = pl.pallas_call(
        _final_body,
        grid=(N // NB,),
        in_specs=[
            pl.BlockSpec((NB, MUL), lambda i: (i, 0)),
            pl.BlockSpec((NB, NATTR), lambda i: (i, 0)),
        ] + t_specs + [
            pl.BlockSpec((MUL, NATTR, MUL), lambda i: (0, 0, 0)),
            pl.BlockSpec((MUL, MUL), lambda i: (0, 0)),
            pl.BlockSpec((HID, 3 * MUL), lambda i: (0, 0)),
            pl.BlockSpec((MUL, MUL), lambda i: (0, 0)),
            pl.BlockSpec((MUL, MUL), lambda i: (0, 0)),
            pl.BlockSpec((MUL, MUL), lambda i: (0, 0)),
        ],
        out_specs=[
            pl.BlockSpec((9, NB, MUL), lambda i: (0, i, 0)),
            pl.BlockSpec((NB, MUL), lambda i: (i, 0)),
        ],
        out_shape=[
            jax.ShapeDtypeStruct((9, N, MUL), jnp.float32),
            jax.ShapeDtypeStruct((N, MUL), jnp.float32),
        ],
    )(node_feat, node_attr, *t_args,
      W_skip, W_lin1, W_mlp3, W2_0, W2_1, W2_2)

    return (jnp.transpose(out9, (1, 2, 0)), sc)


# speedup vs baseline: 1.3265x; 1.1068x over previous
import functools
import math

import jax
import jax.numpy as jnp
from jax import lax
from jax.experimental import pallas as pl
from jax.experimental.pallas import tpu as pltpu
from jax.experimental.pallas import tpu_sc as plsc

MUL = 128
HID = 64
NATTR = 10
NPASS = 5
GW = 2 * HID
AVG_NUM_NEIGHBORS = 32.0
_L_OF_D = (0, 1, 1, 1, 2, 2, 2, 2, 2)


def _mlp_g_body(ed_ref, sh_ref, w0_ref, w1_ref, w2_ref, *g_refs):
    h = jax.nn.silu(jnp.dot(ed_ref[...], w0_ref[...],
                            preferred_element_type=jnp.float32) * (1.0 / math.sqrt(8.0)))
    h = jax.nn.silu(jnp.dot(h, w1_ref[...],
                            preferred_element_type=jnp.float32) * (1.0 / math.sqrt(HID)))
    h = jax.nn.silu(jnp.dot(h, w2_ref[...],
                            preferred_element_type=jnp.float32) * (1.0 / math.sqrt(HID)))
    sh = sh_ref[...]
    for p, ref in enumerate(g_refs):
        cols = []
        for d in range(2):
            dc = 2 * p + d
            cols.append(sh[:, dc:dc + 1] * h if dc < 9 else jnp.zeros_like(h))
        ref[...] = jnp.concatenate(cols, axis=1)


def _build_sc_scatter(E, N):
    n_tiles = 32
    ept = E // n_tiles
    C = 40
    n_chunks = ept // C
    NBUF = 5
    n_outer = n_chunks // NBUF
    rows = (N + 127) // 128 * 128 + 128
    rows_per_tile = rows // 16
    mesh = plsc.VectorSubcoreMesh(core_axis_name="c", subcore_axis_name="s")
    out_t = tuple(jax.ShapeDtypeStruct((2, rows, GW), jnp.float32) for _ in range(NPASS))

    @functools.partial(
        pl.kernel, out_type=out_t, mesh=mesh,
        scratch_types=[pltpu.VMEM((C, GW), jnp.float32)] * NBUF
                      + [pltpu.VMEM((C,), jnp.int32)] * NBUF + [
            pltpu.VMEM_SHARED((rows, GW), jnp.float32),
        ] + [pltpu.SemaphoreType.DMA] * NBUF)
    def sc_scatter(src_hbm, g0_hbm, g1_hbm, g2_hbm, g3_hbm, g4_hbm, zero_hbm,
                   o0, o1, o2, o3, o4, *rest):
        bufs = rest[:NBUF]
        idxs = rest[NBUF:2 * NBUF]
        accum = rest[2 * NBUF]
        sems = rest[2 * NBUF + 1:]
        c = lax.axis_index("c")
        s = lax.axis_index("s")
        wid = c * 16 + s
        tile_base = wid * ept
        row0 = s * rows_per_tile

        for g_hbm, o_hbm in ((g0_hbm, o0), (g1_hbm, o1), (g2_hbm, o2),
                             (g3_hbm, o3), (g4_hbm, o4)):
            pltpu.sync_copy(zero_hbm.at[pl.ds(row0, rows_per_tile)],
                            accum.at[pl.ds(row0, rows_per_tile)])
            plsc.subcore_barrier()

            def fetch(i, b):
                pltpu.async_copy(src_hbm.at[wid, i], idxs[b], sems[b])
                pltpu.async_copy(g_hbm.at[pl.ds(tile_base + i * C, C)],
                                 bufs[b], sems[b])

            for b in range(NBUF):
                fetch(b, b)

            def outer(j, carry):
                for b in range(NBUF):
                    i = j * NBUF + b
                    pltpu.make_async_copy(src_hbm.at[0, 0], idxs[b],
                                          sems[b]).wait()
                    pltpu.make_async_copy(g_hbm.at[pl.ds(0, C)], bufs[b],
                                          sems[b]).wait()
                    pltpu.sync_copy(bufs[b], accum.at[idxs[b]], add=True)

                    @pl.when(j < n_outer - 1)
                    def _():
                        fetch(i + NBUF, b)
                return carry

            lax.fori_loop(0, n_outer, outer, 0)
            plsc.subcore_barrier()
            pltpu.sync_copy(accum.at[pl.ds(row0, rows_per_tile)],
                            o_hbm.at[c, pl.ds(row0, rows_per_tile)])

    return sc_scatter


def _final_body(nf_ref, na_ref, *rest):
    t_refs = rest[:2 * NPASS]
    (wskip_ref, wlin_ref, wmlp3_ref, w20_ref, w21_ref, w22_ref,
     out9_ref, sc_ref) = rest[2 * NPASS:]
    nf = nf_ref[...]
    acc = jnp.zeros_like(nf)
    for v in range(NATTR):
        acc = acc + jnp.dot(nf, wskip_ref[:, v, :],
                            preferred_element_type=jnp.float32) * na_ref[:, v:v + 1]
    sc_ref[...] = acc * (1.0 / math.sqrt(MUL * NATTR))
    x = jnp.dot(nf, wlin_ref[...],
                preferred_element_type=jnp.float32) * (1.0 / math.sqrt(MUL))
    w2s = (w20_ref, w21_ref, w22_ref)
    scale = 1.0 / (math.sqrt(HID) * math.sqrt(MUL) * AVG_NUM_NEIGHBORS)
    for d in range(9):
        p, dl = divmod(d, 2)
        l = _L_OF_D[d]
        ta, tb = t_refs[2 * p], t_refs[2 * p + 1]
        td = (ta[0, :, HID * dl:HID * (dl + 1)]
              + tb[0, :, HID * dl:HID * (dl + 1)])
        m = jnp.dot(td, wmlp3_ref[:, l * MUL:(l + 1) * MUL],
                    preferred_element_type=jnp.float32)
        out9_ref[d] = jnp.dot(x * m, w2s[l][...],
                              preferred_element_type=jnp.float32) * scale


def kernel(node_feat, node_attr, edge_idx, edge_dist_embedding, edge_diff_embedding,
           W_skip, W_lin1, W_mlp0, W_mlp1, W_mlp2, W_mlp3, W2_0, W2_1, W2_2):
    N = node_feat.shape[0]
    E = edge_dist_embedding.shape[0]
    src = edge_idx[:, 0]
    rows = (N + 127) // 128 * 128 + 128

    EB = 4000
    gs = pl.pallas_call(
        _mlp_g_body,
        grid=(E // EB,),
        in_specs=[
            pl.BlockSpec((EB, 8), lambda i: (i, 0)),
            pl.BlockSpec((EB, 9), lambda i: (i, 0)),
            pl.BlockSpec((8, HID), lambda i: (0, 0)),
            pl.BlockSpec((HID, HID), lambda i: (0, 0)),
            pl.BlockSpec((HID, HID), lambda i: (0, 0)),
        ],
        out_specs=[pl.BlockSpec((EB, GW), lambda i: (i, 0))] * NPASS,
        out_shape=[jax.ShapeDtypeStruct((E, GW), jnp.float32)] * NPASS,
    )(edge_dist_embedding, edge_diff_embedding, W_mlp0, W_mlp1, W_mlp2)

    zeros = jnp.zeros((rows, GW), jnp.float32)
    ts = _build_sc_scatter(E, N)(src.reshape(32, -1, 40), *gs, zeros)

    NB = 400
    t_specs = []
    t_args = []
    for t in ts:
        t_specs += [pl.BlockSpec((1, NB, GW), lambda i: (0, i, 0)),
                    pl.BlockSpec((1, NB, GW), lambda i: (1, i, 0))]
        t_args += [t, t]
    out9, sc = pl.pallas_call(
        _final_body,
        grid=(N // NB,),
        in_specs=[
            pl.BlockSpec((NB, MUL), lambda i: (i, 0)),
            pl.BlockSpec((NB, NATTR), lambda i: (i, 0)),
        ] + t_specs + [
            pl.BlockSpec((MUL, NATTR, MUL), lambda i: (0, 0, 0)),
            pl.BlockSpec((MUL, MUL), lambda i: (0, 0)),
            pl.BlockSpec((HID, 3 * MUL), lambda i: (0, 0)),
            pl.BlockSpec((MUL, MUL), lambda i: (0, 0)),
            pl.BlockSpec((MUL, MUL), lambda i: (0, 0)),
            pl.BlockSpec((MUL, MUL), lambda i: (0, 0)),
        ],
        out_specs=[
            pl.BlockSpec((9, NB, MUL), lambda i: (0, i, 0)),
            pl.BlockSpec((NB, MUL), lambda i: (i, 0)),
        ],
        out_shape=[
            jax.ShapeDtypeStruct((9, N, MUL), jnp.float32),
            jax.ShapeDtypeStruct((N, MUL), jnp.float32),
        ],
    )(node_feat, node_attr, *t_args,
      W_skip, W_lin1, W_mlp3, W2_0, W2_1, W2_2)

    return (jnp.transpose(out9, (1, 2, 0)), sc)
